# Initial kernel scaffold; baseline (speedup 1.0000x reference)
import jax
import jax.numpy as jnp
from jax.experimental import pallas as pl


def kernel(pred_loc, pred_bclass, true_loc_vec, true_bclass):
    raise NotImplementedError("write your pallas kernel here")


# speedup vs baseline: 3.0408x; 3.0408x over previous
import jax
import jax.numpy as jnp
from jax.experimental import pallas as pl

_F32_INF_BITS = 0x7F800000


def _ce_kernel(logits_ref, tgt_ref, loss_ref):
    x = logits_ref[0]
    t = tgt_ref[0]
    m = jnp.max(x, axis=0, keepdims=True)
    e = jnp.exp(x - m)
    s = jnp.sum(e, axis=0, keepdims=True)
    lse = m + jnp.log(s)
    cls = jax.lax.broadcasted_iota(jnp.int32, x.shape, 0)
    tl = jnp.sum(jnp.where(cls == t, x, 0.0), axis=0, keepdims=True)
    loss_ref[0] = lse - tl


def _select_kernel(loss_ref, tgt_ref, out_ref):
    loss = loss_ref[:, 0, :]
    tgt = tgt_ref[:, 0, :]
    n = loss.shape[1]

    mask = tgt > 0
    pos_num = jnp.sum(mask.astype(jnp.int32), axis=1, keepdims=True)
    pos_sum = jnp.sum(jnp.where(mask, loss, 0.0), axis=1, keepdims=True)
    con = jnp.where(mask, 0.0, jnp.maximum(loss, 0.0))
    bits = jax.lax.bitcast_convert_type(con, jnp.int32)
    k = jnp.minimum(3 * pos_num, n)

    def body(_, carry):
        lo, hi = carry
        mid = lo + ((hi - lo + 1) >> 1)
        cnt = jnp.sum((bits >= mid).astype(jnp.int32), axis=1, keepdims=True)
        ge = cnt >= k
        return jnp.where(ge, mid, lo), jnp.where(ge, hi, mid - 1)

    lo0 = jnp.zeros_like(k)
    hi0 = jnp.full_like(k, _F32_INF_BITS)
    t_bits, _ = jax.lax.fori_loop(0, 31, body, (lo0, hi0))
    t = jax.lax.bitcast_convert_type(t_bits, jnp.float32)

    gt = bits > t_bits
    cnt_gt = jnp.sum(gt.astype(jnp.int32), axis=1, keepdims=True)
    s_gt = jnp.sum(jnp.where(gt, con, 0.0), axis=1, keepdims=True)
    topk = s_gt + t * (k - cnt_gt).astype(jnp.float32)
    out_ref[...] = pos_sum + jnp.where(k > 0, topk, 0.0)


def kernel(pred_loc, pred_bclass, true_loc_vec, true_bclass):
    del pred_loc, true_loc_vec
    b, c, n = pred_bclass.shape
    tb3 = true_bclass.reshape(b, 1, n)

    nb = 4096
    nt = pl.cdiv(n, nb)
    loss3 = pl.pallas_call(
        _ce_kernel,
        grid=(b, nt),
        in_specs=[
            pl.BlockSpec((1, c, nb), lambda i, j: (i, 0, j)),
            pl.BlockSpec((1, 1, nb), lambda i, j: (i, 0, j)),
        ],
        out_specs=pl.BlockSpec((1, 1, nb), lambda i, j: (i, 0, j)),
        out_shape=jax.ShapeDtypeStruct((b, 1, n), jnp.float32),
    )(pred_bclass, tb3)

    out = pl.pallas_call(
        _select_kernel,
        in_specs=[
            pl.BlockSpec((b, 1, n), lambda: (0, 0, 0)),
            pl.BlockSpec((b, 1, n), lambda: (0, 0, 0)),
        ],
        out_specs=pl.BlockSpec((b, 1), lambda: (0, 0)),
        out_shape=jax.ShapeDtypeStruct((b, 1), jnp.float32),
    )(loss3, tb3)
    return out.reshape(b)


# speedup vs baseline: 4.5686x; 1.5024x over previous
import jax
import jax.numpy as jnp
from jax.experimental import pallas as pl

_F32_INF_BITS = 0x7F800000


def _ce_kernel(logits_ref, tgt_ref, loss_ref):
    x = logits_ref[...]
    t = tgt_ref[...]
    m = jnp.max(x, axis=1, keepdims=True)
    e = jnp.exp(x - m)
    s = jnp.sum(e, axis=1, keepdims=True)
    lse = m + jnp.log(s)
    cls = jax.lax.broadcasted_iota(jnp.int32, x.shape, 1)
    tl = jnp.sum(jnp.where(cls == t, x, 0.0), axis=1, keepdims=True)
    loss_ref[...] = (lse - tl)[:, 0, :]


def _select_kernel(loss_ref, tgt_ref, out_ref):
    loss = loss_ref[...]
    tgt = tgt_ref[...]
    n = loss.shape[1]

    mask = tgt > 0
    pos_num = jnp.sum(mask.astype(jnp.int32), axis=1, keepdims=True)
    pos_sum = jnp.sum(jnp.where(mask, loss, 0.0), axis=1, keepdims=True)
    con = jnp.where(mask, 0.0, jnp.maximum(loss, 0.0))
    bits = jax.lax.bitcast_convert_type(con, jnp.int32)
    k = jnp.minimum(3 * pos_num, n)

    def body(_, carry):
        lo, hi = carry
        mid = lo + ((hi - lo + 1) >> 1)
        cnt = jnp.sum((bits >= mid).astype(jnp.int32), axis=1, keepdims=True)
        ge = cnt >= k
        return jnp.where(ge, mid, lo), jnp.where(ge, hi, mid - 1)

    lo0 = jnp.zeros_like(k)
    hi0 = jnp.full_like(k, _F32_INF_BITS)
    t_bits, _ = jax.lax.fori_loop(0, 31, body, (lo0, hi0))
    t = jax.lax.bitcast_convert_type(t_bits, jnp.float32)

    gt = bits > t_bits
    cnt_gt = jnp.sum(gt.astype(jnp.int32), axis=1, keepdims=True)
    s_gt = jnp.sum(jnp.where(gt, con, 0.0), axis=1, keepdims=True)
    topk = s_gt + t * (k - cnt_gt).astype(jnp.float32)
    out_ref[...] = pos_sum + jnp.where(k > 0, topk, 0.0)


def kernel(pred_loc, pred_bclass, true_loc_vec, true_bclass):
    del pred_loc, true_loc_vec
    b, c, n = pred_bclass.shape
    tb3 = true_bclass.reshape(b, 1, n)

    rb = 8
    nb = 2048
    nt = pl.cdiv(n, nb)
    loss = pl.pallas_call(
        _ce_kernel,
        grid=(b // rb, nt),
        in_specs=[
            pl.BlockSpec((rb, c, nb), lambda i, j: (i, 0, j)),
            pl.BlockSpec((rb, 1, nb), lambda i, j: (i, 0, j)),
        ],
        out_specs=pl.BlockSpec((rb, nb), lambda i, j: (i, j)),
        out_shape=jax.ShapeDtypeStruct((b, n), jnp.float32),
    )(pred_bclass, tb3)

    out = pl.pallas_call(
        _select_kernel,
        in_specs=[
            pl.BlockSpec((b, n), lambda: (0, 0)),
            pl.BlockSpec((b, n), lambda: (0, 0)),
        ],
        out_specs=pl.BlockSpec((b, 1), lambda: (0, 0)),
        out_shape=jax.ShapeDtypeStruct((b, 1), jnp.float32),
    )(loss, true_bclass)
    return out.reshape(b)


# speedup vs baseline: 14.5563x; 3.1861x over previous
import jax
import jax.numpy as jnp
from jax.experimental import pallas as pl

_F32_INF_BITS = 0x7F800000


def _ce_kernel(logits_ref, tgt_ref, loss_ref):
    x = logits_ref[...]
    t = tgt_ref[...]
    m = jnp.max(x, axis=0, keepdims=True)
    e = jnp.exp(x - m)
    s = jnp.sum(e, axis=0, keepdims=True)
    lse = m[0] + jnp.log(s[0])
    cls = jax.lax.broadcasted_iota(jnp.int32, x.shape, 0)
    tl = jnp.sum(jnp.where(cls == t[None], x, 0.0), axis=0)
    loss_ref[...] = lse - tl


def _select_kernel(loss_ref, tgt_ref, out_ref):
    loss = loss_ref[...]
    tgt = tgt_ref[...]
    n = loss.shape[1]

    mask = tgt > 0
    pos_num = jnp.sum(mask.astype(jnp.int32), axis=1, keepdims=True)
    pos_sum = jnp.sum(jnp.where(mask, loss, 0.0), axis=1, keepdims=True)
    con = jnp.where(mask, 0.0, jnp.maximum(loss, 0.0))
    bits = jax.lax.bitcast_convert_type(con, jnp.int32)
    k = jnp.minimum(3 * pos_num, n)

    def body(_, carry):
        lo, hi = carry
        mid = lo + ((hi - lo + 1) >> 1)
        cnt = jnp.sum((bits >= mid).astype(jnp.int32), axis=1, keepdims=True)
        ge = cnt >= k
        return jnp.where(ge, mid, lo), jnp.where(ge, hi, mid - 1)

    lo0 = jnp.zeros_like(k)
    hi0 = jnp.full_like(k, _F32_INF_BITS)
    t_bits, _ = jax.lax.fori_loop(0, 31, body, (lo0, hi0))
    t = jax.lax.bitcast_convert_type(t_bits, jnp.float32)

    gt = bits > t_bits
    cnt_gt = jnp.sum(gt.astype(jnp.int32), axis=1, keepdims=True)
    s_gt = jnp.sum(jnp.where(gt, con, 0.0), axis=1, keepdims=True)
    topk = s_gt + t * (k - cnt_gt).astype(jnp.float32)
    out_ref[...] = pos_sum + jnp.where(k > 0, topk, 0.0)


def kernel(pred_loc, pred_bclass, true_loc_vec, true_bclass):
    del pred_loc, true_loc_vec
    b, c, n = pred_bclass.shape
    pb_t = jnp.transpose(pred_bclass, (1, 0, 2))

    rb = 8
    nb = 4096
    nt = pl.cdiv(n, nb)
    loss = pl.pallas_call(
        _ce_kernel,
        grid=(b // rb, nt),
        in_specs=[
            pl.BlockSpec((c, rb, nb), lambda i, j: (0, i, j)),
            pl.BlockSpec((rb, nb), lambda i, j: (i, j)),
        ],
        out_specs=pl.BlockSpec((rb, nb), lambda i, j: (i, j)),
        out_shape=jax.ShapeDtypeStruct((b, n), jnp.float32),
    )(pb_t, true_bclass)

    out = pl.pallas_call(
        _select_kernel,
        in_specs=[
            pl.BlockSpec((b, n), lambda: (0, 0)),
            pl.BlockSpec((b, n), lambda: (0, 0)),
        ],
        out_specs=pl.BlockSpec((b, 1), lambda: (0, 0)),
        out_shape=jax.ShapeDtypeStruct((b, 1), jnp.float32),
    )(loss, true_bclass)
    return out.reshape(b)
